# Initial kernel scaffold; baseline (speedup 1.0000x reference)
import jax
import jax.numpy as jnp
from jax.experimental import pallas as pl


def kernel(x, cpd_centroid, y, uvp_dim, sigma, enc_nW, enc_nb, enc_eW, enc_eb, eW0, eb0, nW0, nb0, eW1, eb1, nW1, nb1, dec_W, dec_b, edge_index, batch, cell_type):
    raise NotImplementedError("write your pallas kernel here")


# speedup vs baseline: 2.2228x; 2.2228x over previous
import functools

import jax
import jax.numpy as jnp
from jax import lax
from jax.experimental import pallas as pl
from jax.experimental.pallas import tpu as pltpu
from jax.experimental.pallas import tpu_sc as plsc

F32 = jnp.float32
I32 = jnp.int32
H = 64
NC, NS, L = 2, 16, 16
NW = NC * NS
CH = 128
BE = 4096
HI = jax.lax.Precision.HIGHEST


def _cdiv(a, b):
    return (a + b - 1) // b


def _pick_block(n, target):
    best = 8
    for d in range(8, target + 1, 8):
        if n % d == 0:
            best = d
    return best


def _dot(a, b):
    return jax.lax.dot_general(a, b, (((1,), (0,)), ((), ())),
                               precision=HI, preferred_element_type=F32)


def _tc_stats(xe, batch2, nb):
    n = xe.shape[0]
    bn = _pick_block(n, 2048)

    def body(xe_ref, b_ref, o_ref):
        i = pl.program_id(0)

        @pl.when(i == 0)
        def _():
            o_ref[...] = jnp.zeros_like(o_ref)

        xb = xe_ref[...]
        bb = b_ref[...]
        gid = lax.broadcasted_iota(I32, (xb.shape[0], nb), 1)
        oh = (bb == gid).astype(F32)
        sums = _dot(oh.T, xb)
        sqs = _dot(oh.T, xb * xb)
        o_ref[...] += jnp.concatenate([sums, sqs], axis=0)

    return pl.pallas_call(
        body,
        grid=(n // bn,),
        in_specs=[pl.BlockSpec((bn, 8), lambda i: (i, 0)),
                  pl.BlockSpec((bn, 1), lambda i: (i, 0))],
        out_specs=pl.BlockSpec((2 * nb, 8), lambda i: (0, 0)),
        out_shape=jax.ShapeDtypeStruct((2 * nb, 8), F32),
    )(xe, batch2)


def _tc_encode(xe, cpd, batch2, stats, Wn8, bn_, We8, Wp, Ws0, Wr0):
    n = xe.shape[0]
    bn = _pick_block(n, 2048)
    nb = stats.shape[0] // 2

    def body(xe_ref, c_ref, b_ref, st_ref, wn_ref, bn_ref, we_ref, wp_ref,
             ws_ref, wr_ref, h_ref, t1_ref, t2_ref):
        st = st_ref[...]
        sums, sqs = st[:nb], st[nb:]
        cnt = jnp.maximum(sums[:, 7:8], 1.0)
        mean = sums / cnt
        var = jnp.maximum(sqs / cnt - mean * mean, 0.0)
        tot = jnp.sum(sums, axis=0, keepdims=True)
        totsq = jnp.sum(sqs, axis=0, keepdims=True)
        ntot = tot[:, 7:8]
        gmean = tot / ntot
        gstd = jnp.sqrt(jnp.maximum(totsq / ntot - gmean * gmean, 0.0))
        inv_phi = 1.0 / (jnp.sqrt(var) + 1e-8)
        inv_g = 1.0 / (gstd + 1e-8)
        colid = lax.broadcasted_iota(I32, (nb, 8), 1)
        is_phi = colid < 3
        is_g = (colid >= 3) & (colid < 7)
        off = (jnp.where(is_phi, mean, 0.0)
               + jnp.where(is_g, jnp.broadcast_to(gmean, (nb, 8)), 0.0))
        scl = (jnp.where(is_phi, inv_phi, 0.0)
               + jnp.where(is_g, jnp.broadcast_to(inv_g, (nb, 8)), 0.0))

        xb = xe_ref[...]
        bb = b_ref[...]
        gid = lax.broadcasted_iota(I32, (xb.shape[0], nb), 1)
        oh = (bb == gid).astype(F32)
        xn = (xb - _dot(oh, off)) * _dot(oh, scl)
        h0 = jax.nn.relu(_dot(xn, wn_ref[...]) + bn_ref[...])
        q = _dot(xn, we_ref[...]) + _dot(c_ref[...], wp_ref[...])
        h_ref[...] = h0
        t1_ref[...] = jnp.concatenate([q, _dot(h0, ws_ref[...])], axis=1)
        t2_ref[...] = jnp.concatenate([q, _dot(h0, wr_ref[...])], axis=1)

    return pl.pallas_call(
        body,
        grid=(n // bn,),
        in_specs=[pl.BlockSpec((bn, 8), lambda i: (i, 0)),
                  pl.BlockSpec((bn, 2), lambda i: (i, 0)),
                  pl.BlockSpec((bn, 1), lambda i: (i, 0)),
                  pl.BlockSpec(stats.shape, lambda i: (0, 0)),
                  pl.BlockSpec((8, H), lambda i: (0, 0)),
                  pl.BlockSpec((1, H), lambda i: (0, 0)),
                  pl.BlockSpec((8, H), lambda i: (0, 0)),
                  pl.BlockSpec((2, H), lambda i: (0, 0)),
                  pl.BlockSpec((H, H), lambda i: (0, 0)),
                  pl.BlockSpec((H, H), lambda i: (0, 0))],
        out_specs=[pl.BlockSpec((bn, H), lambda i: (i, 0)),
                   pl.BlockSpec((bn, 2 * H), lambda i: (i, 0)),
                   pl.BlockSpec((bn, 2 * H), lambda i: (i, 0))],
        out_shape=[jax.ShapeDtypeStruct((n, H), F32),
                   jax.ShapeDtypeStruct((n, 2 * H), F32),
                   jax.ShapeDtypeStruct((n, 2 * H), F32)],
    )(xe, cpd, batch2, stats, Wn8, bn_, We8, Wp, Ws0, Wr0)


def _sc_gather_enc(T1, T2, sp, rp):
    ep = sp.shape[0]
    per_w = ep // NW
    nch = per_w // CH
    mesh = plsc.VectorSubcoreMesh(core_axis_name="c", subcore_axis_name="s",
                                  num_cores=NC, num_subcores=NS)

    def body(t1_h, t2_h, s_h, r_h, d_o, p_o,
             si, ri, x1, x2, dd, pp, sem):
        w = lax.axis_index("s") * NC + lax.axis_index("c")

        def chunk(j, carry):
            base = w * per_w + j * CH
            pltpu.sync_copy(s_h.at[pl.ds(base, CH)], si)
            pltpu.sync_copy(r_h.at[pl.ds(base, CH)], ri)
            c1 = pltpu.async_copy(t1_h.at[si], x1, sem)
            c2 = pltpu.async_copy(t2_h.at[ri], x2, sem)
            c1.wait(); c2.wait()

            def row(i, c2_):
                for k in range(H // L):
                    sl = pl.ds(L * k, L)
                    s2 = pl.ds(H + L * k, L)
                    dd[i, sl] = x1[i, sl] - x2[i, sl]
                    pp[i, sl] = x1[i, s2] + x2[i, s2]
                return c2_

            lax.fori_loop(0, CH, row, 0)
            pltpu.sync_copy(dd, d_o.at[pl.ds(base, CH)])
            pltpu.sync_copy(pp, p_o.at[pl.ds(base, CH)])
            return carry

        lax.fori_loop(0, nch, chunk, 0)

    return pl.kernel(
        body,
        out_type=(jax.ShapeDtypeStruct((ep, H), F32),
                  jax.ShapeDtypeStruct((ep, H), F32)),
        mesh=mesh,
        scratch_types=[pltpu.VMEM((CH,), I32), pltpu.VMEM((CH,), I32),
                       pltpu.VMEM((CH, 2 * H), F32), pltpu.VMEM((CH, 2 * H), F32),
                       pltpu.VMEM((CH, H), F32), pltpu.VMEM((CH, H), F32),
                       pltpu.SemaphoreType.DMA],
    )(T1, T2, sp, rp)


def _sc_gather_rel(cpd16, sp, rp):
    ep = sp.shape[0]
    per_w = ep // NW
    nch = per_w // CH
    mesh = plsc.VectorSubcoreMesh(core_axis_name="c", subcore_axis_name="s",
                                  num_cores=NC, num_subcores=NS)

    def body(c_h, s_h, r_h, rel_o, si, ri, cs, cr, rr, sem):
        w = lax.axis_index("s") * NC + lax.axis_index("c")

        def chunk(j, carry):
            base = w * per_w + j * CH
            pltpu.sync_copy(s_h.at[pl.ds(base, CH)], si)
            pltpu.sync_copy(r_h.at[pl.ds(base, CH)], ri)
            c1 = pltpu.async_copy(c_h.at[si], cs, sem)
            c2 = pltpu.async_copy(c_h.at[ri], cr, sem)
            c1.wait(); c2.wait()

            def row(i, c2_):
                rr[i, pl.ds(0, L)] = cs[i, pl.ds(0, L)] - cr[i, pl.ds(0, L)]
                return c2_

            lax.fori_loop(0, CH, row, 0)
            pltpu.sync_copy(rr, rel_o.at[pl.ds(base, CH)])
            return carry

        lax.fori_loop(0, nch, chunk, 0)

    return pl.kernel(
        body,
        out_type=jax.ShapeDtypeStruct((ep, L), F32),
        mesh=mesh,
        scratch_types=[pltpu.VMEM((CH,), I32), pltpu.VMEM((CH,), I32),
                       pltpu.VMEM((CH, L), F32), pltpu.VMEM((CH, L), F32),
                       pltpu.VMEM((CH, L), F32),
                       pltpu.SemaphoreType.DMA],
        compiler_params=pltpu.CompilerParams(use_tc_tiling_on_sc=False),
    )(cpd16, sp, rp)


def _sc_gather_pair(T3, sp, rp):
    ep = sp.shape[0]
    per_w = ep // NW
    nch = per_w // CH
    mesh = plsc.VectorSubcoreMesh(core_axis_name="c", subcore_axis_name="s",
                                  num_cores=NC, num_subcores=NS)

    def body(t_h, s_h, r_h, p_o, si, ri, x1, x2, pp, sem):
        w = lax.axis_index("s") * NC + lax.axis_index("c")

        def chunk(j, carry):
            base = w * per_w + j * CH
            pltpu.sync_copy(s_h.at[pl.ds(base, CH)], si)
            pltpu.sync_copy(r_h.at[pl.ds(base, CH)], ri)
            c1 = pltpu.async_copy(t_h.at[si], x1, sem)
            c2 = pltpu.async_copy(t_h.at[ri], x2, sem)
            c1.wait(); c2.wait()

            def row(i, c2_):
                for k in range(H // L):
                    sl = pl.ds(L * k, L)
                    s2 = pl.ds(H + L * k, L)
                    pp[i, sl] = x1[i, sl] + x2[i, s2]
                return c2_

            lax.fori_loop(0, CH, row, 0)
            pltpu.sync_copy(pp, p_o.at[pl.ds(base, CH)])
            return carry

        lax.fori_loop(0, nch, chunk, 0)

    return pl.kernel(
        body,
        out_type=jax.ShapeDtypeStruct((ep, H), F32),
        mesh=mesh,
        scratch_types=[pltpu.VMEM((CH,), I32), pltpu.VMEM((CH,), I32),
                       pltpu.VMEM((CH, 2 * H), F32), pltpu.VMEM((CH, 2 * H), F32),
                       pltpu.VMEM((CH, H), F32),
                       pltpu.SemaphoreType.DMA],
    )(T3, sp, rp)


def _sc_scatter(msg, ridx, zeros, base0, q, nspq):
    ep = msg.shape[0]
    per_t = ep // NS
    nch = per_t // CH
    rows_t = nspq // NS
    mesh = plsc.VectorSubcoreMesh(core_axis_name="c", subcore_axis_name="s",
                                  num_cores=NC, num_subcores=NS)

    def body(m_h, r_h, z_h, o_h, iv, iv2, mv, shared, sem):
        c = lax.axis_index("c")
        s = lax.axis_index("s")
        base_id = base0 + c * q

        pltpu.sync_copy(z_h, shared.at[pl.ds(s * rows_t, rows_t)])
        plsc.subcore_barrier()

        def chunk(j, carry):
            base = s * per_t + j * CH
            pltpu.sync_copy(r_h.at[pl.ds(base, CH)], iv)
            pltpu.sync_copy(m_h.at[pl.ds(base, CH)], mv)

            def fix(k, carry2):
                v = iv[pl.ds(L * k, L)] - base_id
                oob = (v < 0) | (v >= q)
                iv2[pl.ds(L * k, L)] = jnp.where(oob, q, v)
                return carry2

            lax.fori_loop(0, CH // L, fix, 0)
            pltpu.sync_copy(mv, shared.at[iv2], add=True)
            return carry

        lax.fori_loop(0, nch, chunk, 0)
        plsc.subcore_barrier()
        pltpu.sync_copy(shared.at[pl.ds(s * rows_t, rows_t)],
                        o_h.at[pl.ds(c * nspq + s * rows_t, rows_t)])

    return pl.kernel(
        body,
        out_type=jax.ShapeDtypeStruct((NC * nspq, H), F32),
        mesh=mesh,
        scratch_types=[pltpu.VMEM((CH,), I32), pltpu.VMEM((CH,), I32),
                       pltpu.VMEM((CH, H), F32),
                       pltpu.VMEM_SHARED((nspq, H), F32),
                       pltpu.SemaphoreType.DMA],
    )(msg, ridx, zeros)


def _scatter_full(msg, rscat, zeros, n, q, nspq):
    o0 = _sc_scatter(msg, rscat, zeros, 0, q, nspq)
    o1 = _sc_scatter(msg, rscat, zeros, 2 * q, q, nspq)
    return jnp.concatenate([o0[:q], o0[nspq:nspq + q],
                            o1[:q], o1[nspq:nspq + q]], axis=0)[:n]


def _tc_edge0(d, rel, pre0, w9, ebenc, We0, eb0):
    ep = d.shape[0]

    def body(d_ref, r_ref, p_ref, w9_ref, be_ref, w_ref, b_ref, e_ref, m_ref):
        rl = r_ref[...]
        nrm = jnp.sqrt(rl[:, 0:1] * rl[:, 0:1] + rl[:, 1:2] * rl[:, 1:2])
        e0 = jax.nn.relu(d_ref[...] + nrm * w9_ref[...] + be_ref[...])
        e_ref[...] = e0
        m_ref[...] = jax.nn.relu(_dot(e0, w_ref[...]) + p_ref[...] + b_ref[...])

    return pl.pallas_call(
        body,
        grid=(ep // BE,),
        in_specs=[pl.BlockSpec((BE, H), lambda i: (i, 0)),
                  pl.BlockSpec((BE, L), lambda i: (i, 0)),
                  pl.BlockSpec((BE, H), lambda i: (i, 0)),
                  pl.BlockSpec((1, H), lambda i: (0, 0)),
                  pl.BlockSpec((1, H), lambda i: (0, 0)),
                  pl.BlockSpec((H, H), lambda i: (0, 0)),
                  pl.BlockSpec((1, H), lambda i: (0, 0))],
        out_specs=[pl.BlockSpec((BE, H), lambda i: (i, 0)),
                   pl.BlockSpec((BE, H), lambda i: (i, 0))],
        out_shape=[jax.ShapeDtypeStruct((ep, H), F32),
                   jax.ShapeDtypeStruct((ep, H), F32)],
    )(d, rel, pre0, w9, ebenc, We0, eb0)


def _tc_edge1(e0, msg0, pre1, We1, eb1):
    ep = e0.shape[0]

    def body(e_ref, m0_ref, p_ref, w_ref, b_ref, m_ref):
        e1 = e_ref[...] + m0_ref[...]
        m_ref[...] = jax.nn.relu(_dot(e1, w_ref[...]) + p_ref[...] + b_ref[...])

    return pl.pallas_call(
        body,
        grid=(ep // BE,),
        in_specs=[pl.BlockSpec((BE, H), lambda i: (i, 0)),
                  pl.BlockSpec((BE, H), lambda i: (i, 0)),
                  pl.BlockSpec((BE, H), lambda i: (i, 0)),
                  pl.BlockSpec((H, H), lambda i: (0, 0)),
                  pl.BlockSpec((1, H), lambda i: (0, 0))],
        out_specs=pl.BlockSpec((BE, H), lambda i: (i, 0)),
        out_shape=jax.ShapeDtypeStruct((ep, H), F32),
    )(e0, msg0, pre1, We1, eb1)


def _tc_node0(h0, agg, nWh, nWa, nb_, Ws1, Wr1):
    n = h0.shape[0]
    bn = _pick_block(n, 2048)

    def body(h_ref, a_ref, wh_ref, wa_ref, b_ref,
             ws_ref, wr_ref, h1_ref, t3_ref):
        h0b = h_ref[...]
        z = (_dot(h0b, wh_ref[...]) + _dot(a_ref[...], wa_ref[...])
             + b_ref[...])
        h1 = h0b + jax.nn.relu(z)
        h1_ref[...] = h1
        t3_ref[...] = jnp.concatenate(
            [_dot(h1, ws_ref[...]), _dot(h1, wr_ref[...])], axis=1)

    return pl.pallas_call(
        body,
        grid=(n // bn,),
        in_specs=[pl.BlockSpec((bn, H), lambda i: (i, 0)),
                  pl.BlockSpec((bn, H), lambda i: (i, 0)),
                  pl.BlockSpec((H, H), lambda i: (0, 0)),
                  pl.BlockSpec((H, H), lambda i: (0, 0)),
                  pl.BlockSpec((1, H), lambda i: (0, 0)),
                  pl.BlockSpec((H, H), lambda i: (0, 0)),
                  pl.BlockSpec((H, H), lambda i: (0, 0))],
        out_specs=[pl.BlockSpec((bn, H), lambda i: (i, 0)),
                   pl.BlockSpec((bn, 2 * H), lambda i: (i, 0))],
        out_shape=[jax.ShapeDtypeStruct((n, H), F32),
                   jax.ShapeDtypeStruct((n, 2 * H), F32)],
    )(h0, agg, nWh, nWa, nb_, Ws1, Wr1)


def _tc_node1(h1, agg, nWh, nWa, nb_, decW, decb,
              y, cell2, batch2, dimsig):
    n = h1.shape[0]
    bn = _pick_block(n, 2048)
    nb4 = dimsig.shape[0]

    def body(h_ref, a_ref, wh_ref, wa_ref, b_ref,
             dw_ref, db_ref, y_ref, ct_ref, bt_ref, ds_ref, o_ref):
        h1b = h_ref[...]
        z = (_dot(h1b, wh_ref[...]) + _dot(a_ref[...], wa_ref[...])
             + b_ref[...])
        h2 = h1b + jax.nn.relu(z)
        uvp = _dot(h2, dw_ref[...]) + db_ref[...]
        ct = ct_ref[...]
        mask_d = (ct == 6) | (ct == 4) | (ct == 7) | (ct == 8)
        mask_p = ct == 7
        yb = y_ref[...]
        uv = jnp.where(mask_d, yb[:, 0:2], uvp[:, 0:2])
        p = jnp.where(mask_p, 0.0, uvp[:, 2:3])
        uvp2 = jnp.concatenate([uv, p], axis=1)
        bb = bt_ref[...]
        gid = lax.broadcasted_iota(I32, (uvp2.shape[0], nb4), 1)
        oh = (bb == gid).astype(F32)
        o_ref[...] = uvp2 * _dot(oh, ds_ref[...])

    return pl.pallas_call(
        body,
        grid=(n // bn,),
        in_specs=[pl.BlockSpec((bn, H), lambda i: (i, 0)),
                  pl.BlockSpec((bn, H), lambda i: (i, 0)),
                  pl.BlockSpec((H, H), lambda i: (0, 0)),
                  pl.BlockSpec((H, H), lambda i: (0, 0)),
                  pl.BlockSpec((1, H), lambda i: (0, 0)),
                  pl.BlockSpec((H, 3), lambda i: (0, 0)),
                  pl.BlockSpec((1, 3), lambda i: (0, 0)),
                  pl.BlockSpec((bn, 3), lambda i: (i, 0)),
                  pl.BlockSpec((bn, 1), lambda i: (i, 0)),
                  pl.BlockSpec((bn, 1), lambda i: (i, 0)),
                  pl.BlockSpec((nb4, 3), lambda i: (0, 0))],
        out_specs=pl.BlockSpec((bn, 3), lambda i: (i, 0)),
        out_shape=jax.ShapeDtypeStruct((n, 3), F32),
    )(h1, agg, nWh, nWa, nb_, decW, decb, y, cell2, batch2, dimsig)


def kernel(x, cpd_centroid, y, uvp_dim, sigma, enc_nW, enc_nb, enc_eW, enc_eb,
           eW0, eb0, nW0, nb0, eW1, eb1, nW1, nb1, dec_W, dec_b,
           edge_index, batch, cell_type):
    n = x.shape[0]
    e = edge_index.shape[1]
    hh = H // 2

    ep = _cdiv(e, NW * CH) * NW * CH
    q = _cdiv(n, 4)
    nspq = _cdiv(q + 1, NS * 56) * NS * 56
    zeros = jnp.zeros((nspq // NS, H), F32)

    pad = ep - e
    sp = jnp.concatenate([edge_index[0], jnp.zeros((pad,), I32)])
    rp = jnp.concatenate([edge_index[1], jnp.zeros((pad,), I32)])
    rscat = jnp.concatenate([edge_index[1], jnp.full((pad,), n, I32)])

    xe = jnp.concatenate([x, jnp.ones((n, 1), F32)], axis=1)
    cpd16 = jnp.concatenate([cpd_centroid, jnp.zeros((n, L - 2), F32)], axis=1)
    batch2 = batch.reshape(n, 1)
    cell2 = cell_type.reshape(n, 1)

    Wn8 = jnp.concatenate([enc_nW, jnp.zeros((1, H), F32)], axis=0)
    We8 = jnp.concatenate([enc_eW[0:7], jnp.zeros((1, H), F32)], axis=0)
    Wp = enc_eW[7:9]
    w9 = enc_eW[9:10]
    dimsig = uvp_dim * sigma

    stats = _tc_stats(xe, batch2, 4)
    h0, T1, T2 = _tc_encode(xe, cpd_centroid, batch2, stats, Wn8,
                            enc_nb.reshape(1, H), We8, Wp,
                            eW0[H:2 * H], eW0[2 * H:3 * H])

    d, pre0 = _sc_gather_enc(T1, T2, sp, rp)
    rel = _sc_gather_rel(cpd16, sp, rp)
    e0, msg0 = _tc_edge0(d, rel, pre0, w9, enc_eb.reshape(1, H),
                         eW0[0:H], eb0.reshape(1, H))
    agg0 = _scatter_full(msg0, rscat, zeros, n, q, nspq)
    h1, T3 = _tc_node0(h0, agg0, nW0[0:H], nW0[H:2 * H],
                       nb0.reshape(1, H), eW1[H:2 * H], eW1[2 * H:3 * H])

    pre1 = _sc_gather_pair(T3, sp, rp)
    msg1 = _tc_edge1(e0, msg0, pre1, eW1[0:H], eb1.reshape(1, H))
    agg1 = _scatter_full(msg1, rscat, zeros, n, q, nspq)
    out = _tc_node1(h1, agg1, nW1[0:H], nW1[H:2 * H],
                    nb1.reshape(1, H), dec_W, dec_b.reshape(1, 3),
                    y, cell2, batch2, dimsig)
    return out


# speedup vs baseline: 2.3972x; 1.0785x over previous
import functools

import jax
import jax.numpy as jnp
from jax import lax
from jax.experimental import pallas as pl
from jax.experimental.pallas import tpu as pltpu
from jax.experimental.pallas import tpu_sc as plsc

F32 = jnp.float32
I32 = jnp.int32
H = 64
NC, NS, L = 2, 16, 16
NW = NC * NS
CH = 112
BE = 4096
HI = jax.lax.Precision.HIGHEST


def _cdiv(a, b):
    return (a + b - 1) // b


def _pick_block(n, target):
    best = 8
    for d in range(8, target + 1, 8):
        if n % d == 0:
            best = d
    return best


def _dot(a, b):
    return jax.lax.dot_general(a, b, (((1,), (0,)), ((), ())),
                               precision=HI, preferred_element_type=F32)


def _tc_stats(xe, batch2, nb):
    n = xe.shape[0]
    bn = _pick_block(n, 2048)

    def body(xe_ref, b_ref, o_ref):
        i = pl.program_id(0)

        @pl.when(i == 0)
        def _():
            o_ref[...] = jnp.zeros_like(o_ref)

        xb = xe_ref[...]
        bb = b_ref[...]
        gid = lax.broadcasted_iota(I32, (xb.shape[0], nb), 1)
        oh = (bb == gid).astype(F32)
        sums = _dot(oh.T, xb)
        sqs = _dot(oh.T, xb * xb)
        o_ref[...] += jnp.concatenate([sums, sqs], axis=0)

    return pl.pallas_call(
        body,
        grid=(n // bn,),
        in_specs=[pl.BlockSpec((bn, 8), lambda i: (i, 0)),
                  pl.BlockSpec((bn, 1), lambda i: (i, 0))],
        out_specs=pl.BlockSpec((2 * nb, 8), lambda i: (0, 0)),
        out_shape=jax.ShapeDtypeStruct((2 * nb, 8), F32),
    )(xe, batch2)


def _tc_encode(xe, cpd, batch2, stats, Wn8, bn_, We8, Wp, Ws0, Wr0):
    n = xe.shape[0]
    bn = _pick_block(n, 2048)
    nb = stats.shape[0] // 2

    def body(xe_ref, c_ref, b_ref, st_ref, wn_ref, bn_ref, we_ref, wp_ref,
             ws_ref, wr_ref, h_ref, t1_ref, t2_ref):
        st = st_ref[...]
        sums, sqs = st[:nb], st[nb:]
        cnt = jnp.maximum(sums[:, 7:8], 1.0)
        mean = sums / cnt
        var = jnp.maximum(sqs / cnt - mean * mean, 0.0)
        tot = jnp.sum(sums, axis=0, keepdims=True)
        totsq = jnp.sum(sqs, axis=0, keepdims=True)
        ntot = tot[:, 7:8]
        gmean = tot / ntot
        gstd = jnp.sqrt(jnp.maximum(totsq / ntot - gmean * gmean, 0.0))
        inv_phi = 1.0 / (jnp.sqrt(var) + 1e-8)
        inv_g = 1.0 / (gstd + 1e-8)
        colid = lax.broadcasted_iota(I32, (nb, 8), 1)
        is_phi = colid < 3
        is_g = (colid >= 3) & (colid < 7)
        off = (jnp.where(is_phi, mean, 0.0)
               + jnp.where(is_g, jnp.broadcast_to(gmean, (nb, 8)), 0.0))
        scl = (jnp.where(is_phi, inv_phi, 0.0)
               + jnp.where(is_g, jnp.broadcast_to(inv_g, (nb, 8)), 0.0))

        xb = xe_ref[...]
        bb = b_ref[...]
        gid = lax.broadcasted_iota(I32, (xb.shape[0], nb), 1)
        oh = (bb == gid).astype(F32)
        xn = (xb - _dot(oh, off)) * _dot(oh, scl)
        h0 = jax.nn.relu(_dot(xn, wn_ref[...]) + bn_ref[...])
        q = _dot(xn, we_ref[...]) + _dot(c_ref[...], wp_ref[...])
        h_ref[...] = h0
        t1_ref[...] = jnp.concatenate([q, _dot(h0, ws_ref[...])], axis=1)
        t2_ref[...] = jnp.concatenate([q, _dot(h0, wr_ref[...])], axis=1)

    return pl.pallas_call(
        body,
        grid=(n // bn,),
        in_specs=[pl.BlockSpec((bn, 8), lambda i: (i, 0)),
                  pl.BlockSpec((bn, 2), lambda i: (i, 0)),
                  pl.BlockSpec((bn, 1), lambda i: (i, 0)),
                  pl.BlockSpec(stats.shape, lambda i: (0, 0)),
                  pl.BlockSpec((8, H), lambda i: (0, 0)),
                  pl.BlockSpec((1, H), lambda i: (0, 0)),
                  pl.BlockSpec((8, H), lambda i: (0, 0)),
                  pl.BlockSpec((2, H), lambda i: (0, 0)),
                  pl.BlockSpec((H, H), lambda i: (0, 0)),
                  pl.BlockSpec((H, H), lambda i: (0, 0))],
        out_specs=[pl.BlockSpec((bn, H), lambda i: (i, 0)),
                   pl.BlockSpec((bn, 2 * H), lambda i: (i, 0)),
                   pl.BlockSpec((bn, 2 * H), lambda i: (i, 0))],
        out_shape=[jax.ShapeDtypeStruct((n, H), F32),
                   jax.ShapeDtypeStruct((n, 2 * H), F32),
                   jax.ShapeDtypeStruct((n, 2 * H), F32)],
    )(xe, cpd, batch2, stats, Wn8, bn_, We8, Wp, Ws0, Wr0)


def _sc_ring_gather(tA, tB, sp, rp, mode):
    ep = sp.shape[0]
    per_w = ep // NW
    nch = per_w // CH
    assert nch % 2 == 0
    tw = tA.shape[1]
    ow = tw if mode == "rel" else H
    two_out = mode == "enc"
    mesh = plsc.VectorSubcoreMesh(core_axis_name="c", subcore_axis_name="s",
                                  num_cores=NC, num_subcores=NS)

    def body(ta_h, tb_h, s_h, r_h, *rest):
        if two_out:
            d_o, p_o = rest[0], rest[1]
            rest = rest[2:]
        else:
            d_o = p_o = rest[0]
            rest = rest[1:]
        (si0, si1, ri0, ri1, x10, x11, x20, x21,
         dd0, dd1, pp0, pp1, gs0, gs1, is0, is1, ws0, ws1) = rest
        si = (si0, si1); ri = (ri0, ri1)
        x1 = (x10, x11); x2 = (x20, x21)
        dd = (dd0, dd1); pp = (pp0, pp1)
        gs = (gs0, gs1); isem = (is0, is1); ws = (ws0, ws1)
        w = lax.axis_index("s") * NC + lax.axis_index("c")
        base0 = w * per_w

        pltpu.sync_copy(s_h.at[pl.ds(base0, CH)], si0)
        pltpu.sync_copy(r_h.at[pl.ds(base0, CH)], ri0)
        pltpu.async_copy(ta_h.at[si0], x10, gs0)
        pltpu.async_copy(tb_h.at[ri0], x20, gs0)
        pltpu.async_copy(s_h.at[pl.ds(base0 + CH, CH)], si1, is1)
        pltpu.async_copy(r_h.at[pl.ds(base0 + CH, CH)], ri1, is1)

        def compute(b):
            def row(i, cc):
                if mode == "enc":
                    for k in range(H // L):
                        sl = pl.ds(L * k, L)
                        s2 = pl.ds(H + L * k, L)
                        dd[b][i, sl] = x1[b][i, sl] - x2[b][i, sl]
                        pp[b][i, sl] = x1[b][i, s2] + x2[b][i, s2]
                elif mode == "pair":
                    for k in range(H // L):
                        sl = pl.ds(L * k, L)
                        s2 = pl.ds(H + L * k, L)
                        pp[b][i, sl] = x1[b][i, sl] + x2[b][i, s2]
                else:
                    pp[b][i, pl.ds(0, L)] = (x1[b][i, pl.ds(0, L)]
                                             - x2[b][i, pl.ds(0, L)])
                return cc

            lax.fori_loop(0, CH, row, 0)

        def outer(jj, carry):
            for b in (0, 1):
                j = jj * 2 + b
                bo = 1 - b

                @pl.when(j >= 2)
                def _():
                    if two_out:
                        pltpu.make_async_copy(
                            dd[b], d_o.at[pl.ds(base0, CH)], ws[b]).wait()
                    pltpu.make_async_copy(
                        pp[b], p_o.at[pl.ds(base0, CH)], ws[b]).wait()

                pltpu.make_async_copy(ta_h.at[si[b]], x1[b], gs[b]).wait()
                pltpu.make_async_copy(tb_h.at[ri[b]], x2[b], gs[b]).wait()

                @pl.when(j + 1 < nch)
                def _():
                    off = base0 + (j + 1) * CH
                    pltpu.make_async_copy(
                        s_h.at[pl.ds(off, CH)], si[bo], isem[bo]).wait()
                    pltpu.make_async_copy(
                        r_h.at[pl.ds(off, CH)], ri[bo], isem[bo]).wait()
                    pltpu.async_copy(ta_h.at[si[bo]], x1[bo], gs[bo])
                    pltpu.async_copy(tb_h.at[ri[bo]], x2[bo], gs[bo])

                @pl.when(j + 2 < nch)
                def _():
                    off = base0 + (j + 2) * CH
                    pltpu.async_copy(s_h.at[pl.ds(off, CH)], si[b], isem[b])
                    pltpu.async_copy(r_h.at[pl.ds(off, CH)], ri[b], isem[b])

                compute(b)
                off = base0 + j * CH
                if two_out:
                    pltpu.async_copy(dd[b], d_o.at[pl.ds(off, CH)], ws[b])
                pltpu.async_copy(pp[b], p_o.at[pl.ds(off, CH)], ws[b])
            return carry

        lax.fori_loop(0, nch // 2, outer, 0)
        for b in (0, 1):
            if two_out:
                pltpu.make_async_copy(
                    dd[b], d_o.at[pl.ds(base0, CH)], ws[b]).wait()
            pltpu.make_async_copy(pp[b], p_o.at[pl.ds(base0, CH)], ws[b]).wait()

    outs = (jax.ShapeDtypeStruct((ep, H), F32),) * 2 if two_out else         jax.ShapeDtypeStruct((ep, ow), F32)
    scratch = [pltpu.VMEM((CH,), I32)] * 4 +         [pltpu.VMEM((CH, tw), F32)] * 4 +         [pltpu.VMEM((CH, ow), F32)] * 4 +         [pltpu.SemaphoreType.DMA] * 6
    params = (pltpu.CompilerParams(use_tc_tiling_on_sc=False)
              if mode == "rel" else None)
    return pl.kernel(body, out_type=outs, mesh=mesh, scratch_types=scratch,
                     compiler_params=params)(tA, tB, sp, rp)


def _sc_gather_pair_sb(T3, sp, rp):
    ep = sp.shape[0]
    per_w = ep // NW
    nch = per_w // CH
    mesh = plsc.VectorSubcoreMesh(core_axis_name="c", subcore_axis_name="s",
                                  num_cores=NC, num_subcores=NS)

    def body(t_h, s_h, r_h, p_o, si, ri, x1, x2, pp, sem):
        w = lax.axis_index("s") * NC + lax.axis_index("c")

        def chunk(j, carry):
            base = w * per_w + j * CH
            pltpu.sync_copy(s_h.at[pl.ds(base, CH)], si)
            pltpu.sync_copy(r_h.at[pl.ds(base, CH)], ri)
            c1 = pltpu.async_copy(t_h.at[si], x1, sem)
            c2 = pltpu.async_copy(t_h.at[ri], x2, sem)
            c1.wait(); c2.wait()

            def row(i, c2_):
                for k in range(H // L):
                    sl = pl.ds(L * k, L)
                    s2 = pl.ds(H + L * k, L)
                    pp[i, sl] = x1[i, sl] + x2[i, s2]
                return c2_

            lax.fori_loop(0, CH, row, 0)
            pltpu.sync_copy(pp, p_o.at[pl.ds(base, CH)])
            return carry

        lax.fori_loop(0, nch, chunk, 0)

    return pl.kernel(
        body,
        out_type=jax.ShapeDtypeStruct((ep, H), F32),
        mesh=mesh,
        scratch_types=[pltpu.VMEM((CH,), I32), pltpu.VMEM((CH,), I32),
                       pltpu.VMEM((CH, 2 * H), F32), pltpu.VMEM((CH, 2 * H), F32),
                       pltpu.VMEM((CH, H), F32),
                       pltpu.SemaphoreType.DMA],
    )(T3, sp, rp)


def _sc_gather_enc(T1, T2, sp, rp):
    return _sc_ring_gather(T1, T2, sp, rp, "enc")


def _sc_gather_rel(cpd16, sp, rp):
    return _sc_ring_gather(cpd16, cpd16, sp, rp, "rel")


def _sc_gather_pair(T3, sp, rp):
    return _sc_ring_gather(T3, T3, sp, rp, "pair")


def _sc_scatter(msg, ridx, zeros, base0, q, nspq):
    ep = msg.shape[0]
    per_t = ep // NS
    SCH = 128
    nch = per_t // SCH
    rows_t = nspq // NS
    mesh = plsc.VectorSubcoreMesh(core_axis_name="c", subcore_axis_name="s",
                                  num_cores=NC, num_subcores=NS)

    def body(m_h, r_h, z_h, o_h, iv, iw, mv, shared, sem):
        c = lax.axis_index("c")
        s = lax.axis_index("s")
        base_id = base0 + c * q

        pltpu.sync_copy(z_h, shared.at[pl.ds(s * rows_t, rows_t)])
        plsc.subcore_barrier()

        def chunk(j, carry):
            base = s * per_t + j * SCH
            pltpu.sync_copy(r_h.at[pl.ds(base, SCH)], iv)
            pltpu.sync_copy(m_h.at[pl.ds(base, SCH)], mv)

            def fix(k, carry2):
                v = iv[pl.ds(L * k, L)] - base_id
                oob = (v < 0) | (v >= q)
                iw[pl.ds(L * k, L)] = jnp.where(oob, q, v)
                return carry2

            lax.fori_loop(0, SCH // L, fix, 0)
            pltpu.sync_copy(mv, shared.at[iw], add=True)
            return carry

        lax.fori_loop(0, nch, chunk, 0)
        plsc.subcore_barrier()
        pltpu.sync_copy(shared.at[pl.ds(s * rows_t, rows_t)],
                        o_h.at[pl.ds(c * nspq + s * rows_t, rows_t)])

    return pl.kernel(
        body,
        out_type=jax.ShapeDtypeStruct((NC * nspq, H), F32),
        mesh=mesh,
        scratch_types=[pltpu.VMEM((SCH,), I32), pltpu.VMEM((SCH,), I32),
                       pltpu.VMEM((SCH, H), F32),
                       pltpu.VMEM_SHARED((nspq, H), F32),
                       pltpu.SemaphoreType.DMA],
    )(msg, ridx, zeros)


def _scatter_full(msg, rscat, zeros, n, q, nspq):
    o0 = _sc_scatter(msg, rscat, zeros, 0, q, nspq)
    o1 = _sc_scatter(msg, rscat, zeros, 2 * q, q, nspq)
    return jnp.concatenate([o0[:q], o0[nspq:nspq + q],
                            o1[:q], o1[nspq:nspq + q]], axis=0)[:n]


def _tc_edge0(d, rel, pre0, w9, ebenc, We0, eb0):
    ep = d.shape[0]

    def body(d_ref, r_ref, p_ref, w9_ref, be_ref, w_ref, b_ref, e_ref, m_ref):
        rl = r_ref[...]
        nrm = jnp.sqrt(rl[:, 0:1] * rl[:, 0:1] + rl[:, 1:2] * rl[:, 1:2])
        e0 = jax.nn.relu(d_ref[...] + nrm * w9_ref[...] + be_ref[...])
        e_ref[...] = e0
        m_ref[...] = jax.nn.relu(_dot(e0, w_ref[...]) + p_ref[...] + b_ref[...])

    return pl.pallas_call(
        body,
        grid=(ep // BE,),
        in_specs=[pl.BlockSpec((BE, H), lambda i: (i, 0)),
                  pl.BlockSpec((BE, L), lambda i: (i, 0)),
                  pl.BlockSpec((BE, H), lambda i: (i, 0)),
                  pl.BlockSpec((1, H), lambda i: (0, 0)),
                  pl.BlockSpec((1, H), lambda i: (0, 0)),
                  pl.BlockSpec((H, H), lambda i: (0, 0)),
                  pl.BlockSpec((1, H), lambda i: (0, 0))],
        out_specs=[pl.BlockSpec((BE, H), lambda i: (i, 0)),
                   pl.BlockSpec((BE, H), lambda i: (i, 0))],
        out_shape=[jax.ShapeDtypeStruct((ep, H), F32),
                   jax.ShapeDtypeStruct((ep, H), F32)],
    )(d, rel, pre0, w9, ebenc, We0, eb0)


def _tc_edge1(e0, msg0, pre1, We1, eb1):
    ep = e0.shape[0]

    def body(e_ref, m0_ref, p_ref, w_ref, b_ref, m_ref):
        e1 = e_ref[...] + m0_ref[...]
        m_ref[...] = jax.nn.relu(_dot(e1, w_ref[...]) + p_ref[...] + b_ref[...])

    return pl.pallas_call(
        body,
        grid=(ep // BE,),
        in_specs=[pl.BlockSpec((BE, H), lambda i: (i, 0)),
                  pl.BlockSpec((BE, H), lambda i: (i, 0)),
                  pl.BlockSpec((BE, H), lambda i: (i, 0)),
                  pl.BlockSpec((H, H), lambda i: (0, 0)),
                  pl.BlockSpec((1, H), lambda i: (0, 0))],
        out_specs=pl.BlockSpec((BE, H), lambda i: (i, 0)),
        out_shape=jax.ShapeDtypeStruct((ep, H), F32),
    )(e0, msg0, pre1, We1, eb1)


def _tc_node0(h0, agg, nWh, nWa, nb_, Ws1, Wr1):
    n = h0.shape[0]
    bn = _pick_block(n, 2048)

    def body(h_ref, a_ref, wh_ref, wa_ref, b_ref,
             ws_ref, wr_ref, h1_ref, t3_ref):
        h0b = h_ref[...]
        z = (_dot(h0b, wh_ref[...]) + _dot(a_ref[...], wa_ref[...])
             + b_ref[...])
        h1 = h0b + jax.nn.relu(z)
        h1_ref[...] = h1
        t3_ref[...] = jnp.concatenate(
            [_dot(h1, ws_ref[...]), _dot(h1, wr_ref[...])], axis=1)

    return pl.pallas_call(
        body,
        grid=(n // bn,),
        in_specs=[pl.BlockSpec((bn, H), lambda i: (i, 0)),
                  pl.BlockSpec((bn, H), lambda i: (i, 0)),
                  pl.BlockSpec((H, H), lambda i: (0, 0)),
                  pl.BlockSpec((H, H), lambda i: (0, 0)),
                  pl.BlockSpec((1, H), lambda i: (0, 0)),
                  pl.BlockSpec((H, H), lambda i: (0, 0)),
                  pl.BlockSpec((H, H), lambda i: (0, 0))],
        out_specs=[pl.BlockSpec((bn, H), lambda i: (i, 0)),
                   pl.BlockSpec((bn, 2 * H), lambda i: (i, 0))],
        out_shape=[jax.ShapeDtypeStruct((n, H), F32),
                   jax.ShapeDtypeStruct((n, 2 * H), F32)],
    )(h0, agg, nWh, nWa, nb_, Ws1, Wr1)


def _tc_node1(h1, agg, nWh, nWa, nb_, decW, decb,
              y, cell2, batch2, dimsig):
    n = h1.shape[0]
    bn = _pick_block(n, 2048)
    nb4 = dimsig.shape[0]

    def body(h_ref, a_ref, wh_ref, wa_ref, b_ref,
             dw_ref, db_ref, y_ref, ct_ref, bt_ref, ds_ref, o_ref):
        h1b = h_ref[...]
        z = (_dot(h1b, wh_ref[...]) + _dot(a_ref[...], wa_ref[...])
             + b_ref[...])
        h2 = h1b + jax.nn.relu(z)
        uvp = _dot(h2, dw_ref[...]) + db_ref[...]
        ct = ct_ref[...]
        mask_d = (ct == 6) | (ct == 4) | (ct == 7) | (ct == 8)
        mask_p = ct == 7
        yb = y_ref[...]
        uv = jnp.where(mask_d, yb[:, 0:2], uvp[:, 0:2])
        p = jnp.where(mask_p, 0.0, uvp[:, 2:3])
        uvp2 = jnp.concatenate([uv, p], axis=1)
        bb = bt_ref[...]
        gid = lax.broadcasted_iota(I32, (uvp2.shape[0], nb4), 1)
        oh = (bb == gid).astype(F32)
        o_ref[...] = uvp2 * _dot(oh, ds_ref[...])

    return pl.pallas_call(
        body,
        grid=(n // bn,),
        in_specs=[pl.BlockSpec((bn, H), lambda i: (i, 0)),
                  pl.BlockSpec((bn, H), lambda i: (i, 0)),
                  pl.BlockSpec((H, H), lambda i: (0, 0)),
                  pl.BlockSpec((H, H), lambda i: (0, 0)),
                  pl.BlockSpec((1, H), lambda i: (0, 0)),
                  pl.BlockSpec((H, 3), lambda i: (0, 0)),
                  pl.BlockSpec((1, 3), lambda i: (0, 0)),
                  pl.BlockSpec((bn, 3), lambda i: (i, 0)),
                  pl.BlockSpec((bn, 1), lambda i: (i, 0)),
                  pl.BlockSpec((bn, 1), lambda i: (i, 0)),
                  pl.BlockSpec((nb4, 3), lambda i: (0, 0))],
        out_specs=pl.BlockSpec((bn, 3), lambda i: (i, 0)),
        out_shape=jax.ShapeDtypeStruct((n, 3), F32),
    )(h1, agg, nWh, nWa, nb_, decW, decb, y, cell2, batch2, dimsig)


def kernel(x, cpd_centroid, y, uvp_dim, sigma, enc_nW, enc_nb, enc_eW, enc_eb,
           eW0, eb0, nW0, nb0, eW1, eb1, nW1, nb1, dec_W, dec_b,
           edge_index, batch, cell_type):
    n = x.shape[0]
    e = edge_index.shape[1]
    hh = H // 2

    ep = _cdiv(e, NW * CH) * NW * CH
    q = _cdiv(n, 4)
    nspq = _cdiv(q + 1, NS * 56) * NS * 56
    zeros = jnp.zeros((nspq // NS, H), F32)

    pad = ep - e
    sp = jnp.concatenate([edge_index[0], jnp.zeros((pad,), I32)])
    rp = jnp.concatenate([edge_index[1], jnp.zeros((pad,), I32)])
    rscat = jnp.concatenate([edge_index[1], jnp.full((pad,), n, I32)])

    xe = jnp.concatenate([x, jnp.ones((n, 1), F32)], axis=1)
    cpd16 = jnp.concatenate([cpd_centroid, jnp.zeros((n, L - 2), F32)], axis=1)
    batch2 = batch.reshape(n, 1)
    cell2 = cell_type.reshape(n, 1)

    Wn8 = jnp.concatenate([enc_nW, jnp.zeros((1, H), F32)], axis=0)
    We8 = jnp.concatenate([enc_eW[0:7], jnp.zeros((1, H), F32)], axis=0)
    Wp = enc_eW[7:9]
    w9 = enc_eW[9:10]
    dimsig = uvp_dim * sigma

    stats = _tc_stats(xe, batch2, 4)
    h0, T1, T2 = _tc_encode(xe, cpd_centroid, batch2, stats, Wn8,
                            enc_nb.reshape(1, H), We8, Wp,
                            eW0[H:2 * H], eW0[2 * H:3 * H])

    d, pre0 = _sc_gather_enc(T1, T2, sp, rp)
    rel = _sc_gather_rel(cpd16, sp, rp)
    e0, msg0 = _tc_edge0(d, rel, pre0, w9, enc_eb.reshape(1, H),
                         eW0[0:H], eb0.reshape(1, H))
    agg0 = _scatter_full(msg0, rscat, zeros, n, q, nspq)
    h1, T3 = _tc_node0(h0, agg0, nW0[0:H], nW0[H:2 * H],
                       nb0.reshape(1, H), eW1[H:2 * H], eW1[2 * H:3 * H])

    pre1 = _sc_gather_pair_sb(T3, sp, rp)
    msg1 = _tc_edge1(e0, msg0, pre1, eW1[0:H], eb1.reshape(1, H))
    agg1 = _scatter_full(msg1, rscat, zeros, n, q, nspq)
    out = _tc_node1(h1, agg1, nW1[0:H], nW1[H:2 * H],
                    nb1.reshape(1, H), dec_W, dec_b.reshape(1, 3),
                    y, cell2, batch2, dimsig)
    return out
